# baseline (speedup 1.0000x reference)
import functools

import jax
import jax.numpy as jnp
from jax import lax
from jax.experimental import pallas as pl
from jax.experimental.pallas import tpu as pltpu
from jax.experimental.pallas import tpu_sc as plsc

_TOKENS = 2048
_DMODEL = 2048
_NEXP = 8

_NC = 2
_NS = 16
_NW = _NC * _NS
_TPW = _TOKENS // _NW
_CH = 16
_NCHUNK = _TPW // _CH


_NBUF = 2


def _route(e_t, fire, lo=0, hi=_NEXP):
    if hi - lo == 1:
        fire(lo)
        return
    mid = (lo + hi) // 2

    @pl.when(e_t < mid)
    def _():
        _route(e_t, fire, lo, mid)

    @pl.when(e_t >= mid)
    def _():
        _route(e_t, fire, mid, hi)


def _combine_body(e0, e1, e2, e3, e4, e5, e6, e7, gate_hbm, out_hbm,
                  gate_v, rows_v0, rows_v1, gsem, osem0, osem1):
    experts = (e0, e1, e2, e3, e4, e5, e6, e7)
    bufs = (rows_v0, rows_v1)
    osems = (osem0, osem1)
    sid = lax.axis_index("s")
    wid = sid * _NC + lax.axis_index("c")
    base = wid * _TPW

    pltpu.sync_copy(gate_hbm.at[pl.ds(base * _NEXP, _TPW * _NEXP)], gate_v)

    lanes = jnp.arange(_CH, dtype=jnp.int32)

    def pair(p, carry):
        for b in range(_NBUF):
            c = p * _NBUF + b
            rows_v = bufs[b].at[sid]
            osem = osems[b]
            cbase = c * _CH

            flat_ids = (cbase + lanes) * _NEXP
            best = plsc.load_gather(gate_v, [flat_ids])
            bidx = jnp.zeros((_CH,), jnp.int32)
            for e in range(1, _NEXP):
                v = plsc.load_gather(gate_v, [flat_ids + e])
                m = v > best
                best = jnp.where(m, v, best)
                bidx = jnp.where(m, e, bidx)

            @pl.when(c >= _NBUF)
            def _():
                pltpu.make_async_copy(
                    rows_v, out_hbm.at[pl.ds(base, _CH)], osem
                ).wait()

            for t in range(_CH):
                e_t = bidx[t]
                tok = base + cbase + t

                def fire(e, t=t, tok=tok, rows_v=rows_v):
                    pltpu.async_copy(
                        experts[e].at[pl.ds(tok, 1)],
                        rows_v.at[pl.ds(t, 1)],
                        gsem,
                    )

                _route(e_t, fire)
            pltpu.make_async_copy(
                e0.at[pl.ds(0, _CH)], rows_v, gsem
            ).wait()
            pltpu.async_copy(rows_v, out_hbm.at[pl.ds(base + cbase, _CH)], osem)
        return carry

    lax.fori_loop(0, _NCHUNK // _NBUF, pair, 0)
    for b in range(_NBUF):
        pltpu.make_async_copy(
            bufs[b].at[sid], out_hbm.at[pl.ds(base, _CH)], osems[b]
        ).wait()


@jax.jit
def _combine(e0, e1, e2, e3, e4, e5, e6, e7, gate_logits):
    mesh = plsc.VectorSubcoreMesh(
        core_axis_name="c", subcore_axis_name="s",
        num_cores=_NC, num_subcores=_NS,
    )
    f = pl.kernel(
        _combine_body,
        out_type=jax.ShapeDtypeStruct((_TOKENS, _DMODEL), jnp.float32),
        mesh=mesh,
        scratch_types=[
            pltpu.VMEM((_TPW * _NEXP,), jnp.float32),
            pltpu.VMEM_SHARED((_NS, _CH, _DMODEL), jnp.float32),
            pltpu.VMEM_SHARED((_NS, _CH, _DMODEL), jnp.float32),
            pltpu.SemaphoreType.DMA,
            pltpu.SemaphoreType.DMA,
            pltpu.SemaphoreType.DMA,
        ],
        compiler_params=pltpu.CompilerParams(needs_layout_passes=False),
        name="top1_combine_sc",
    )
    return f(e0, e1, e2, e3, e4, e5, e6, e7, gate_logits.reshape(-1))


def kernel(expert_0, expert_1, expert_2, expert_3, expert_4, expert_5,
           expert_6, expert_7, gate_logits):
    return _combine(expert_0, expert_1, expert_2, expert_3, expert_4,
                    expert_5, expert_6, expert_7, gate_logits)


# speedup vs baseline: 1.0336x; 1.0336x over previous
import functools

import jax
import jax.numpy as jnp
from jax import lax
from jax.experimental import pallas as pl
from jax.experimental.pallas import tpu as pltpu
from jax.experimental.pallas import tpu_sc as plsc

_TOKENS = 2048
_DMODEL = 2048
_NEXP = 8

_NC = 2
_NS = 16
_NW = _NC * _NS
_TPW = _TOKENS // _NW
_CH = 16
_NCHUNK = _TPW // _CH


_NBUF = 2


def _route(e_t, fire, lo=0, hi=_NEXP):
    if hi - lo == 1:
        fire(lo)
        return
    mid = (lo + hi) // 2

    @pl.when(e_t < mid)
    def _():
        _route(e_t, fire, lo, mid)

    @pl.when(e_t >= mid)
    def _():
        _route(e_t, fire, mid, hi)


def _combine_body(e0, e1, e2, e3, e4, e5, e6, e7, gate_hbm, out_hbm,
                  gate_v, rows_v0, rows_v1, gsem0, gsem1, osem0, osem1):
    experts = (e0, e1, e2, e3, e4, e5, e6, e7)
    bufs = (rows_v0, rows_v1)
    gsems = (gsem0, gsem1)
    osems = (osem0, osem1)
    sid = lax.axis_index("s")
    wid = sid * _NC + lax.axis_index("c")
    base = wid * _TPW

    pltpu.sync_copy(gate_hbm.at[pl.ds(base * _NEXP, _TPW * _NEXP)], gate_v)

    lanes = jnp.arange(_CH, dtype=jnp.int32)

    _H = _CH // 2

    def fire_half(bidx, c, h, rows_v, gsem):
        for t in range(h * _H, (h + 1) * _H):
            e_t = bidx[t]
            tok = base + c * _CH + t

            def fire(e, t=t, tok=tok):
                pltpu.async_copy(
                    experts[e].at[pl.ds(tok, 1)],
                    rows_v.at[pl.ds(t, 1)],
                    gsem,
                )

            _route(e_t, fire)

    def drain_half(rows_v, gsem):
        pltpu.make_async_copy(e0.at[pl.ds(0, _H)], rows_v.at[pl.ds(0, _H)],
                              gsem).wait()

    def pair(p, carry):
        bidxs = []
        for b in range(_NBUF):
            c = p * _NBUF + b
            flat_ids = (c * _CH + lanes) * _NEXP
            best = plsc.load_gather(gate_v, [flat_ids])
            bidx = jnp.zeros((_CH,), jnp.int32)
            for e in range(1, _NEXP):
                v = plsc.load_gather(gate_v, [flat_ids + e])
                m = v > best
                best = jnp.where(m, v, best)
                bidx = jnp.where(m, e, bidx)
            bidxs.append(bidx)

        c0 = p * _NBUF
        c1 = c0 + 1
        rv0 = bufs[0].at[sid]
        rv1 = bufs[1].at[sid]

        @pl.when(p >= 1)
        def _():
            pltpu.make_async_copy(
                rv0, out_hbm.at[pl.ds(base, _CH)], osems[0]
            ).wait()
        fire_half(bidxs[0], c0, 0, rv0, gsems[0])
        fire_half(bidxs[0], c0, 1, rv0, gsems[1])

        @pl.when(p >= 1)
        def _():
            pltpu.make_async_copy(
                rv1, out_hbm.at[pl.ds(base, _CH)], osems[1]
            ).wait()
        drain_half(rv0, gsems[0])
        fire_half(bidxs[1], c1, 0, rv1, gsems[0])
        drain_half(rv0, gsems[1])
        fire_half(bidxs[1], c1, 1, rv1, gsems[1])
        pltpu.async_copy(rv0, out_hbm.at[pl.ds(base + c0 * _CH, _CH)],
                         osems[0])
        drain_half(rv1, gsems[0])
        drain_half(rv1, gsems[1])
        pltpu.async_copy(rv1, out_hbm.at[pl.ds(base + c1 * _CH, _CH)],
                         osems[1])
        return carry

    lax.fori_loop(0, _NCHUNK // _NBUF, pair, 0)
    for b in range(_NBUF):
        pltpu.make_async_copy(
            bufs[b].at[sid], out_hbm.at[pl.ds(base, _CH)], osems[b]
        ).wait()


@jax.jit
def _combine(e0, e1, e2, e3, e4, e5, e6, e7, gate_logits):
    mesh = plsc.VectorSubcoreMesh(
        core_axis_name="c", subcore_axis_name="s",
        num_cores=_NC, num_subcores=_NS,
    )
    f = pl.kernel(
        _combine_body,
        out_type=jax.ShapeDtypeStruct((_TOKENS, _DMODEL), jnp.float32),
        mesh=mesh,
        scratch_types=[
            pltpu.VMEM((_TPW * _NEXP,), jnp.float32),
            pltpu.VMEM_SHARED((_NS, _CH, _DMODEL), jnp.float32),
            pltpu.VMEM_SHARED((_NS, _CH, _DMODEL), jnp.float32),
            pltpu.SemaphoreType.DMA,
            pltpu.SemaphoreType.DMA,
            pltpu.SemaphoreType.DMA,
            pltpu.SemaphoreType.DMA,
        ],
        compiler_params=pltpu.CompilerParams(needs_layout_passes=False),
        name="top1_combine_sc",
    )
    return f(e0, e1, e2, e3, e4, e5, e6, e7, gate_logits.reshape(-1))


def kernel(expert_0, expert_1, expert_2, expert_3, expert_4, expert_5,
           expert_6, expert_7, gate_logits):
    return _combine(expert_0, expert_1, expert_2, expert_3, expert_4,
                    expert_5, expert_6, expert_7, gate_logits)
